# baseline (speedup 1.0000x reference)
import functools

import jax
import jax.numpy as jnp
from jax import lax
from jax.experimental import pallas as pl
from jax.experimental.pallas import tpu as pltpu
from jax.experimental.pallas import tpu_sc as plsc

EMB = 64
PADW = 128
BATCH = 16384
OUT = 2
NW = 32
BPW = BATCH // NW
HB = BPW // 2
CH = 128

_mesh = plsc.VectorSubcoreMesh(core_axis_name="c", subcore_axis_name="s")


@functools.partial(
    pl.kernel,
    out_type=jax.ShapeDtypeStruct((OUT, BATCH), jnp.float32),
    mesh=_mesh,
    compiler_params=pltpu.CompilerParams(
        needs_layout_passes=False, use_tc_tiling_on_sc=True),
    scratch_types=[
        pltpu.VMEM((2 * BPW,), jnp.int32),
        pltpu.VMEM((2 * HB, PADW), jnp.float32),
        pltpu.VMEM((OUT, EMB), jnp.float32),
        pltpu.VMEM((16,), jnp.float32),
        pltpu.VMEM((OUT, BPW), jnp.float32),
        pltpu.SemaphoreType.DMA,
    ],
)
def _srn2vec_sc(xf_hbm, tbl_hbm, wt_hbm, b_hbm, out_hbm,
                idx_v, rows_v, wt_v, b_v, out_v, sem):
    wid = lax.axis_index("s") * 2 + lax.axis_index("c")
    pltpu.sync_copy(xf_hbm.at[pl.ds(wid * BPW, BPW)], idx_v.at[pl.ds(0, BPW)])
    pltpu.sync_copy(xf_hbm.at[pl.ds(BATCH + wid * BPW, BPW)],
                    idx_v.at[pl.ds(BPW, BPW)])
    pltpu.sync_copy(wt_hbm, wt_v)
    pltpu.sync_copy(b_hbm, b_v)

    wt0 = [wt_v[0, pl.ds(k * 16, 16)] for k in range(EMB // 16)]
    wt1 = [wt_v[1, pl.ds(k * 16, 16)] for k in range(EMB // 16)]
    bvec = b_v[...]
    b0 = bvec[0]
    b1 = bvec[1]
    lanes = lax.iota(jnp.int32, 16)
    col0 = jnp.zeros((16,), jnp.int32)
    col1 = col0 + 1

    for h in range(2):
        copies = []
        for c in range(HB // CH):
            copies.append(pltpu.async_copy(
                tbl_hbm.at[idx_v.at[pl.ds(h * HB + c * CH, CH)]],
                rows_v.at[pl.ds(c * CH, CH), :], sem))
            copies.append(pltpu.async_copy(
                tbl_hbm.at[idx_v.at[pl.ds(BPW + h * HB + c * CH, CH)]],
                rows_v.at[pl.ds(HB + c * CH, CH), :], sem))
        for cp in copies:
            cp.wait()

        def group_body(g, carry):
            y0 = jnp.zeros((16,), jnp.float32)
            y1 = jnp.zeros((16,), jnp.float32)
            for j in range(16):
                b = g * 16 + j
                s0 = jnp.float32(0)
                s1 = jnp.float32(0)
                t0 = jnp.zeros((16,), jnp.float32)
                t1 = jnp.zeros((16,), jnp.float32)
                for k in range(EMB // 16):
                    va = rows_v[b, pl.ds(k * 16, 16)]
                    vb = rows_v[HB + b, pl.ds(k * 16, 16)]
                    p = va * vb
                    t0 = t0 + p * wt0[k]
                    t1 = t1 + p * wt1[k]
                s0 = jnp.sum(t0)
                s1 = jnp.sum(t1)
                y0 = jnp.where(lanes == j, s0, y0)
                y1 = jnp.where(lanes == j, s1, y1)
            y0 = 1.0 / (1.0 + jnp.exp(-(y0 + b0)))
            y1 = 1.0 / (1.0 + jnp.exp(-(y1 + b1)))
            bidx = h * HB + g * 16 + lanes
            plsc.store_scatter(out_v, [col0, bidx], y0)
            plsc.store_scatter(out_v, [col1, bidx], y1)
            return carry

        lax.fori_loop(0, HB // 16, group_body, 0)

    pltpu.sync_copy(out_v, out_hbm.at[:, pl.ds(wid * BPW, BPW)])


def kernel(x, table, W_out, b_out):
    xf = x.T.reshape(-1)
    tblp = jnp.pad(table, ((0, 0), (0, PADW - EMB)))
    wt = W_out.T
    bp = jnp.zeros((16,), jnp.float32).at[:OUT].set(b_out)
    return _srn2vec_sc(xf, tblp, wt, bp).T


# speedup vs baseline: 1.3841x; 1.3841x over previous
import functools

import jax
import jax.numpy as jnp
from jax import lax
from jax.experimental import pallas as pl
from jax.experimental.pallas import tpu as pltpu
from jax.experimental.pallas import tpu_sc as plsc

EMB = 64
BATCH = 16384
OUT = 2
NW = 32
BPW = BATCH // NW
GS = 16
NG = BPW // GS
HALF = 2 * GS

_mesh = plsc.VectorSubcoreMesh(core_axis_name="c", subcore_axis_name="s")


def _fire_group(g, idx_v, tbl_hbm, buf_v, base, sem):
    iv_a = idx_v[pl.ds(g * GS, 16)]
    iv_b = idx_v[pl.ds(BPW + g * GS, 16)]
    for j in range(GS):
        ra = iv_a[j]
        rb = iv_b[j]
        pltpu.async_copy(
            tbl_hbm.at[pl.ds(ra, 1), :],
            buf_v.at[pl.ds(base + j, 1), :], sem)
        pltpu.async_copy(
            tbl_hbm.at[pl.ds(rb, 1), :],
            buf_v.at[pl.ds(base + GS + j, 1), :], sem)


def _drain_group(tbl_hbm, buf_v, sem):
    pltpu.make_async_copy(
        tbl_hbm.at[pl.ds(0, 32), :], buf_v.at[pl.ds(0, 32), :], sem).wait()


def _compute_group(g, buf_v, base, wt0, wt1, b0, b1, out_v,
                   lanes, col0, col1):
    rows_a = base + lanes
    rows_b = base + GS + lanes
    acc0 = jnp.zeros((16,), jnp.float32)
    acc1 = jnp.zeros((16,), jnp.float32)
    for d in range(EMB):
        dvec = jnp.full((16,), d, jnp.int32)
        c0 = plsc.load_gather(buf_v, [rows_a, dvec])
        c1 = plsc.load_gather(buf_v, [rows_b, dvec])
        p = c0 * c1
        acc0 = acc0 + p * wt0[d // 16][d % 16]
        acc1 = acc1 + p * wt1[d // 16][d % 16]
    y0 = 1.0 / (1.0 + jnp.exp(-(acc0 + b0)))
    y1 = 1.0 / (1.0 + jnp.exp(-(acc1 + b1)))
    bidx = g * GS + lanes
    plsc.store_scatter(out_v, [bidx, col0], y0)
    plsc.store_scatter(out_v, [bidx, col1], y1)


@functools.partial(
    pl.kernel,
    out_type=jax.ShapeDtypeStruct((BATCH, OUT), jnp.float32),
    mesh=_mesh,
    compiler_params=pltpu.CompilerParams(
        needs_layout_passes=False, use_tc_tiling_on_sc=True),
    scratch_types=[
        pltpu.VMEM((2 * BPW,), jnp.int32),
        pltpu.VMEM((2 * HALF, EMB), jnp.float32),
        pltpu.VMEM((OUT, EMB), jnp.float32),
        pltpu.VMEM((16,), jnp.float32),
        pltpu.VMEM((BPW, OUT), jnp.float32),
        pltpu.SemaphoreType.DMA,
    ],
)
def _srn2vec_sc(xf_hbm, tbl_hbm, wt_hbm, b_hbm, out_hbm,
                idx_v, buf_v, wt_v, b_v, out_v, sem):
    wid = lax.axis_index("s") * 2 + lax.axis_index("c")
    pltpu.sync_copy(xf_hbm.at[pl.ds(wid * BPW, BPW)], idx_v.at[pl.ds(0, BPW)])
    pltpu.sync_copy(xf_hbm.at[pl.ds(BATCH + wid * BPW, BPW)],
                    idx_v.at[pl.ds(BPW, BPW)])
    pltpu.sync_copy(wt_hbm, wt_v)
    pltpu.sync_copy(b_hbm, b_v)

    wt0 = [wt_v[0, pl.ds(k * 16, 16)] for k in range(EMB // 16)]
    wt1 = [wt_v[1, pl.ds(k * 16, 16)] for k in range(EMB // 16)]
    bvec = b_v[...]
    b0 = bvec[0]
    b1 = bvec[1]
    lanes = lax.iota(jnp.int32, 16)
    col0 = jnp.zeros((16,), jnp.int32)
    col1 = col0 + 1

    _fire_group(0, idx_v, tbl_hbm, buf_v, 0, sem)

    def group_loop(g, carry):
        nbase = lax.rem(g + 1, 2) * HALF
        base = lax.rem(g, 2) * HALF

        @pl.when(g < NG - 1)
        def _():
            _fire_group(g + 1, idx_v, tbl_hbm, buf_v, nbase, sem)

        _drain_group(tbl_hbm, buf_v, sem)
        _compute_group(g, buf_v, base, wt0, wt1, b0, b1, out_v,
                       lanes, col0, col1)
        return carry

    lax.fori_loop(0, NG, group_loop, 0)

    pltpu.sync_copy(out_v, out_hbm.at[pl.ds(wid * BPW, BPW), :])


def kernel(x, table, W_out, b_out):
    xf = x.T.reshape(-1)
    wt = W_out.T
    bp = jnp.zeros((16,), jnp.float32).at[:OUT].set(b_out)
    return _srn2vec_sc(xf, table, wt, bp)
